# baseline (speedup 1.0000x reference)
import functools

import jax
import jax.numpy as jnp
from jax import lax
from jax.experimental import pallas as pl
from jax.experimental.pallas import tpu as pltpu
from jax.experimental.pallas import tpu_sc as plsc

N_NODES = 10000
D = 128
DH = D // 2
N_EDGES = 320000

NC = 2
NS = 16
E_PER_TILE = N_EDGES // NS
CHUNK = 40
N_CHUNKS = E_PER_TILE // CHUNK
K = 5
NG = N_CHUNKS // K
GW = K * CHUNK
N_PAD = 10240
ROWS_PER_TILE = N_PAD // NS
CNT_W = 8
L = 16


def _sc_segment_sum(xr, src, dst, zeros_feat, zeros_cnt, ones):
    mesh = plsc.VectorSubcoreMesh(core_axis_name="c", subcore_axis_name="s")

    @functools.partial(
        pl.kernel,
        out_type=(
            jax.ShapeDtypeStruct((N_PAD, D), jnp.float32),
            jax.ShapeDtypeStruct((N_PAD, D), jnp.float32),
        ),
        mesh=mesh,
        scratch_types=[
            pltpu.VMEM((N_CHUNKS, CHUNK), jnp.int32),
            pltpu.VMEM((N_CHUNKS, CHUNK), jnp.int32),
            pltpu.VMEM((K, CHUNK, DH), jnp.float32),
            pltpu.VMEM((K, CHUNK, DH), jnp.float32),
            pltpu.VMEM((CHUNK, CNT_W), jnp.float32),
            pltpu.VMEM_SHARED((N_PAD, DH), jnp.float32),
            pltpu.VMEM_SHARED((N_PAD, CNT_W), jnp.float32),
            pltpu.SemaphoreType.DMA,
            pltpu.SemaphoreType.DMA,
            pltpu.SemaphoreType.DMA,
        ],
        compiler_params=pltpu.CompilerParams(use_tc_tiling_on_sc=False),
    )
    def k(xr_hbm, src_hbm, dst_hbm, zf_hbm, zc_hbm, ones_hbm,
          sum_out, cnt_out, sidx_v, didx_v, rows0_v, rows1_v, ones_v,
          acc_sh, cnt_sh, gsem, ssem, osem):
        c = lax.axis_index("c")
        s = lax.axis_index("s")
        r0 = s * ROWS_PER_TILE
        e0 = s * E_PER_TILE
        cv = jnp.zeros((L,), jnp.int32) + c

        pltpu.sync_copy(zf_hbm.at[pl.ds(r0, ROWS_PER_TILE)],
                        acc_sh.at[pl.ds(r0, ROWS_PER_TILE)])
        pltpu.sync_copy(zc_hbm.at[pl.ds(r0, ROWS_PER_TILE)],
                        cnt_sh.at[pl.ds(r0, ROWS_PER_TILE)])
        pltpu.sync_copy(src_hbm.at[s], sidx_v)
        pltpu.sync_copy(dst_hbm.at[s], didx_v)
        pltpu.sync_copy(ones_hbm, ones_v)
        plsc.subcore_barrier()

        xrv = xr_hbm.at[pl.ds(c, 2 * N_NODES - 1)]

        def start_gathers(g, buf):
            for j in range(K):
                pltpu.async_copy(
                    xrv.at[sidx_v.at[g * K + j]],
                    buf.at[j], gsem)

        def wait_gathers(g, buf):
            for j in range(K):
                pltpu.make_async_copy(
                    xrv.at[sidx_v.at[g * K + j]],
                    buf.at[j], gsem).wait()

        def start_scatters(g, buf):
            for j in range(K):
                pltpu.async_copy(
                    buf.at[j],
                    acc_sh.at[didx_v.at[g * K + j]],
                    ssem, add=True)

            @pl.when((g < NG // 2) == (c == 0))
            def _():
                for j in range(K):
                    pltpu.async_copy(
                        ones_v,
                        cnt_sh.at[didx_v.at[g * K + j]],
                        osem, add=True)

        def wait_scatters(g, buf):
            for j in range(K):
                pltpu.make_async_copy(
                    buf.at[j],
                    acc_sh.at[didx_v.at[g * K + j]],
                    ssem).wait()

        start_gathers(0, rows0_v)

        def body(p, carry):
            a = 2 * p

            @pl.when(p > 0)
            def _():
                wait_scatters(a - 1, rows1_v)

            start_gathers(a + 1, rows1_v)
            wait_gathers(a, rows0_v)
            start_scatters(a, rows0_v)

            @pl.when(p + 1 < NG // 2)
            def _():
                wait_scatters(a, rows0_v)
                start_gathers(a + 2, rows0_v)

            wait_gathers(a + 1, rows1_v)
            start_scatters(a + 1, rows1_v)
            return carry

        lax.fori_loop(0, NG // 2, body, 0)
        wait_scatters(NG - 2, rows0_v)
        wait_scatters(NG - 1, rows1_v)

        def drain(g, carry):
            for j in range(K):
                pltpu.make_async_copy(
                    ones_v,
                    cnt_sh.at[didx_v.at[g * K + j]],
                    osem).wait()
            return carry

        lax.fori_loop(0, NG // 2, drain, 0)
        plsc.subcore_barrier()

        pltpu.sync_copy(acc_sh.at[pl.ds(r0, ROWS_PER_TILE)],
                        sum_out.at[pl.ds(r0, ROWS_PER_TILE),
                                   pl.ds(c * DH, DH)])
        pltpu.sync_copy(cnt_sh.at[pl.ds(r0, ROWS_PER_TILE)],
                        cnt_out.at[pl.ds(r0, ROWS_PER_TILE),
                                   pl.ds(c * CNT_W, CNT_W)])

    return k(xr, src, dst, zeros_feat, zeros_cnt, ones)


def _tc_self(x, wrT, bl):
    BLK = 1000
    grid = (N_NODES // BLK,)

    def body(x_ref, wrT_ref, bl_ref, xr_ref):
        xr_ref[...] = (jnp.dot(x_ref[...], wrT_ref[...],
                               preferred_element_type=jnp.float32)
                       + bl_ref[...])

    return pl.pallas_call(
        body,
        grid=grid,
        in_specs=[
            pl.BlockSpec((BLK, D), lambda i: (i, 0)),
            pl.BlockSpec((D, D), lambda i: (0, 0)),
            pl.BlockSpec((1, D), lambda i: (0, 0)),
        ],
        out_specs=[pl.BlockSpec((BLK, D), lambda i: (i, 0))],
        out_shape=[jax.ShapeDtypeStruct((N_NODES, D), jnp.float32)],
    )(x, wrT, bl)[0]


def _tc_dense(summed, counts, xself, wlT, woT, bo):
    BLK = 1000
    grid = (N_NODES // BLK,)

    def body(sum_ref, cnt_ref, xs_ref, wlT_ref, woT_ref,
             bo_ref, out_ref, h_ref):
        cnt = cnt_ref[:, 0:1] + cnt_ref[:, CNT_W:CNT_W + 1]
        mean = sum_ref[...] / jnp.maximum(cnt, 1.0)
        h = (jnp.dot(mean, wlT_ref[...], preferred_element_type=jnp.float32)
             + xs_ref[...])
        out = (jnp.dot(h, woT_ref[...], preferred_element_type=jnp.float32)
               + bo_ref[...])
        h_ref[...] = h
        out_ref[...] = out

    return pl.pallas_call(
        body,
        grid=grid,
        in_specs=[
            pl.BlockSpec((BLK, D), lambda i: (i, 0)),
            pl.BlockSpec((BLK, D), lambda i: (i, 0)),
            pl.BlockSpec((BLK, D), lambda i: (i, 0)),
            pl.BlockSpec((D, D), lambda i: (0, 0)),
            pl.BlockSpec((D, D), lambda i: (0, 0)),
            pl.BlockSpec((1, D), lambda i: (0, 0)),
        ],
        out_specs=[
            pl.BlockSpec((BLK, D), lambda i: (i, 0)),
            pl.BlockSpec((BLK, D), lambda i: (i, 0)),
        ],
        out_shape=[
            jax.ShapeDtypeStruct((N_NODES, D), jnp.float32),
            jax.ShapeDtypeStruct((N_NODES, D), jnp.float32),
        ],
    )(summed, counts, xself, wlT, woT, bo)


def kernel(x, edge_index, W_l, b_l, W_r, W_out, b_out):
    ei = edge_index.astype(jnp.int32)
    s1 = ei[0]
    src = (s1 + s1).reshape(NS, N_CHUNKS, CHUNK)
    dst = ei[1].reshape(NS, N_CHUNKS, CHUNK)
    xr = x.reshape(2 * N_NODES, DH)
    zeros_feat = jnp.zeros((N_PAD, DH), jnp.float32)
    zeros_cnt = jnp.zeros((N_PAD, CNT_W), jnp.float32)
    ones = jnp.ones((CHUNK, CNT_W), jnp.float32)

    summed, counts = _sc_segment_sum(xr, src, dst, zeros_feat, zeros_cnt,
                                     ones)
    xself = _tc_self(x, W_r.T, b_l.reshape(1, D))
    out, h = _tc_dense(summed, counts, xself,
                       W_l.T, W_out.T, b_out.reshape(1, D))
    return (out, h)


# speedup vs baseline: 1.0111x; 1.0111x over previous
import functools

import jax
import jax.numpy as jnp
from jax import lax
from jax.experimental import pallas as pl
from jax.experimental.pallas import tpu as pltpu
from jax.experimental.pallas import tpu_sc as plsc

N_NODES = 10000
D = 128
DH = D // 2
N_EDGES = 320000

NC = 2
NS = 16
E_PER_TILE = N_EDGES // NS
CHUNK = 40
N_CHUNKS = E_PER_TILE // CHUNK
K = 5
NG = N_CHUNKS // K
GW = K * CHUNK
N_PAD = 10240
ROWS_PER_TILE = N_PAD // NS
CNT_W = 8
L = 16


def _sc_segment_sum(xr, src, dst, zeros_feat, zeros_cnt, ones):
    mesh = plsc.VectorSubcoreMesh(core_axis_name="c", subcore_axis_name="s")

    @functools.partial(
        pl.kernel,
        out_type=(
            jax.ShapeDtypeStruct((N_PAD, D), jnp.float32),
            jax.ShapeDtypeStruct((N_PAD, D), jnp.float32),
        ),
        mesh=mesh,
        scratch_types=[
            pltpu.VMEM((N_CHUNKS, CHUNK), jnp.int32),
            pltpu.VMEM((N_CHUNKS, CHUNK), jnp.int32),
            pltpu.VMEM((K, CHUNK, DH), jnp.float32),
            pltpu.VMEM((K, CHUNK, DH), jnp.float32),
            pltpu.VMEM((CHUNK, CNT_W), jnp.float32),
            pltpu.VMEM_SHARED((N_PAD, DH), jnp.float32),
            pltpu.VMEM_SHARED((N_PAD, CNT_W), jnp.float32),
            pltpu.SemaphoreType.DMA,
            pltpu.SemaphoreType.DMA,
            pltpu.SemaphoreType.DMA,
        ],
        compiler_params=pltpu.CompilerParams(use_tc_tiling_on_sc=False),
    )
    def k(xr_hbm, src_hbm, dst_hbm, zf_hbm, zc_hbm, ones_hbm,
          sum_out, cnt_out, sidx_v, didx_v, rows0_v, rows1_v, ones_v,
          acc_sh, cnt_sh, gsem, ssem, osem):
        c = lax.axis_index("c")
        s = lax.axis_index("s")
        r0 = s * ROWS_PER_TILE
        e0 = s * E_PER_TILE
        cv = jnp.zeros((L,), jnp.int32) + c

        pltpu.sync_copy(zf_hbm.at[pl.ds(r0, ROWS_PER_TILE)],
                        acc_sh.at[pl.ds(r0, ROWS_PER_TILE)])
        pltpu.sync_copy(zc_hbm.at[pl.ds(r0, ROWS_PER_TILE)],
                        cnt_sh.at[pl.ds(r0, ROWS_PER_TILE)])
        pltpu.sync_copy(src_hbm.at[s], sidx_v)
        pltpu.sync_copy(dst_hbm.at[s], didx_v)
        pltpu.sync_copy(ones_hbm, ones_v)
        plsc.subcore_barrier()

        xrv = xr_hbm.at[pl.ds(c, 2 * N_NODES - 1)]

        def start_gathers(g, buf):
            for j in range(K):
                pltpu.async_copy(
                    xrv.at[sidx_v.at[g * K + j]],
                    buf.at[j], gsem)

        def wait_gathers(g, buf):
            for j in range(K):
                pltpu.make_async_copy(
                    xrv.at[sidx_v.at[g * K + j]],
                    buf.at[j], gsem).wait()

        def start_scatters(g, buf):
            for j in range(K):
                pltpu.async_copy(
                    buf.at[j],
                    acc_sh.at[didx_v.at[g * K + j]],
                    ssem, add=True)

            @pl.when((g < NG // 2) == (c == 0))
            def _():
                for j in range(K):
                    pltpu.async_copy(
                        ones_v,
                        cnt_sh.at[didx_v.at[g * K + j]],
                        osem, add=True)

        def wait_scatters(g, buf):
            for j in range(K):
                pltpu.make_async_copy(
                    buf.at[j],
                    acc_sh.at[didx_v.at[g * K + j]],
                    ssem).wait()

        start_gathers(0, rows0_v)

        def body(p, carry):
            a = 2 * p

            @pl.when(p > 0)
            def _():
                wait_scatters(a - 1, rows1_v)

            start_gathers(a + 1, rows1_v)
            wait_gathers(a, rows0_v)
            start_scatters(a, rows0_v)

            @pl.when(p + 1 < NG // 2)
            def _():
                wait_scatters(a, rows0_v)
                start_gathers(a + 2, rows0_v)

            wait_gathers(a + 1, rows1_v)
            start_scatters(a + 1, rows1_v)
            return carry

        lax.fori_loop(0, NG // 2, body, 0)
        wait_scatters(NG - 2, rows0_v)
        wait_scatters(NG - 1, rows1_v)

        def drain(g, carry):
            for j in range(K):
                pltpu.make_async_copy(
                    ones_v,
                    cnt_sh.at[didx_v.at[g * K + j]],
                    osem).wait()
            return carry

        lax.fori_loop(0, NG // 2, drain, 0)
        plsc.subcore_barrier()

        pltpu.sync_copy(acc_sh.at[pl.ds(r0, ROWS_PER_TILE)],
                        sum_out.at[pl.ds(r0, ROWS_PER_TILE),
                                   pl.ds(c * DH, DH)])
        pltpu.sync_copy(cnt_sh.at[pl.ds(r0, ROWS_PER_TILE)],
                        cnt_out.at[pl.ds(r0, ROWS_PER_TILE),
                                   pl.ds(c * CNT_W, CNT_W)])

    return k(xr, src, dst, zeros_feat, zeros_cnt, ones)


def _tc_dense(summed, counts, x, wlT, wrT, woT, bl, bo):
    BLK = 1000
    grid = (N_NODES // BLK,)

    def body(sum_ref, cnt_ref, x_ref, wlT_ref, wrT_ref, woT_ref,
             bl_ref, bo_ref, out_ref, h_ref):
        cnt = cnt_ref[:, 0:1] + cnt_ref[:, CNT_W:CNT_W + 1]
        mean = sum_ref[...] / jnp.maximum(cnt, 1.0)
        h = (jnp.dot(mean, wlT_ref[...], preferred_element_type=jnp.float32)
             + jnp.dot(x_ref[...], wrT_ref[...],
                       preferred_element_type=jnp.float32)
             + bl_ref[...])
        out = (jnp.dot(h, woT_ref[...], preferred_element_type=jnp.float32)
               + bo_ref[...])
        h_ref[...] = h
        out_ref[...] = out

    return pl.pallas_call(
        body,
        grid=grid,
        in_specs=[
            pl.BlockSpec((BLK, D), lambda i: (i, 0)),
            pl.BlockSpec((BLK, D), lambda i: (i, 0)),
            pl.BlockSpec((BLK, D), lambda i: (i, 0)),
            pl.BlockSpec((D, D), lambda i: (0, 0)),
            pl.BlockSpec((D, D), lambda i: (0, 0)),
            pl.BlockSpec((D, D), lambda i: (0, 0)),
            pl.BlockSpec((1, D), lambda i: (0, 0)),
            pl.BlockSpec((1, D), lambda i: (0, 0)),
        ],
        out_specs=[
            pl.BlockSpec((BLK, D), lambda i: (i, 0)),
            pl.BlockSpec((BLK, D), lambda i: (i, 0)),
        ],
        out_shape=[
            jax.ShapeDtypeStruct((N_NODES, D), jnp.float32),
            jax.ShapeDtypeStruct((N_NODES, D), jnp.float32),
        ],
    )(summed, counts, x, wlT, wrT, woT, bl, bo)


def kernel(x, edge_index, W_l, b_l, W_r, W_out, b_out):
    ei = edge_index.astype(jnp.int32)
    s1 = ei[0]
    src = (s1 + s1).reshape(NS, N_CHUNKS, CHUNK)
    dst = ei[1].reshape(NS, N_CHUNKS, CHUNK)
    xr = x.reshape(2 * N_NODES, DH)
    zeros_feat = jnp.zeros((N_PAD, DH), jnp.float32)
    zeros_cnt = jnp.zeros((N_PAD, CNT_W), jnp.float32)
    ones = jnp.ones((CHUNK, CNT_W), jnp.float32)

    summed, counts = _sc_segment_sum(xr, src, dst, zeros_feat, zeros_cnt,
                                     ones)
    out, h = _tc_dense(summed, counts, x,
                       W_l.T, W_r.T, W_out.T,
                       b_l.reshape(1, D), b_out.reshape(1, D))
    return (out, h)
